# Initial kernel scaffold; baseline (speedup 1.0000x reference)
import jax
import jax.numpy as jnp
from jax.experimental import pallas as pl


def kernel(addition_logits, deletion_logits, edge_candidate_idx, edge_index):
    raise NotImplementedError("write your pallas kernel here")


# speedup vs baseline: 4.3905x; 4.3905x over previous
import jax
import jax.numpy as jnp
from jax.experimental import pallas as pl
from jax.experimental.pallas import tpu as pltpu

_G = 64
_NCAND = 1024
_NEDGE = 2048
_K = 32
_INT_MIN = -2**31


def _sortkey(x):
    b = jax.lax.bitcast_convert_type(x, jnp.int32)
    return jnp.where(b >= 0, b, b ^ jnp.int32(0x7FFFFFFF))


def _kth_largest(key, k):
    S = jnp.full((key.shape[0], 1), _INT_MIN, jnp.int32)
    for bit in range(31, -1, -1):
        cand = (S ^ _INT_MIN) if bit == 31 else (S | jnp.int32(1 << bit))
        cnt = jnp.sum((key >= cand).astype(jnp.int32), axis=1, keepdims=True)
        S = jnp.where(cnt >= k, cand, S)
    return S


def _body(add_ref, del_ref, addw_ref, delw_ref):
    d = del_ref[:]
    dkey = ~_sortkey(d)
    Td = _kth_largest(dkey, _K)
    delw_ref[:] = jnp.where(dkey >= Td, 0.0, 1.0).astype(jnp.float32)

    a = add_ref[:]
    akey = _sortkey(a)
    Ta = _kth_largest(akey, _K)
    m = jnp.max(a, axis=1, keepdims=True)
    p = jnp.exp(a - m)
    s = jnp.sum(p, axis=1, keepdims=True)
    w = jnp.minimum((_K * p) / s, 1.0)
    addw_ref[:] = jnp.where(akey >= Ta, w, 0.0).astype(jnp.float32)


def kernel(addition_logits, deletion_logits, edge_candidate_idx, edge_index):
    add = addition_logits.reshape(_G, _NCAND)
    dele = deletion_logits.reshape(_G, _NEDGE)
    addw, delw = pl.pallas_call(
        _body,
        out_shape=[
            jax.ShapeDtypeStruct((_G, _NCAND), jnp.float32),
            jax.ShapeDtypeStruct((_G, _NEDGE), jnp.float32),
        ],
    )(add, dele)
    merged_edge_weight = jnp.concatenate(
        [delw.reshape(-1), addw.reshape(-1)])
    merged_edge_index = jnp.concatenate(
        [edge_index, edge_candidate_idx.T], axis=1)
    return merged_edge_index, merged_edge_weight
